# baseline (speedup 1.0000x reference)
import functools

import jax
import jax.numpy as jnp
import numpy as np
from jax import lax
from jax.experimental import pallas as pl
from jax.experimental.pallas import tpu as pltpu
from jax.experimental.pallas import tpu_sc as plsc

N = 10000
E = 320000
D = 128

NC = 2
NS = 16
L = 16
NW = NC * NS

C = 96
GB = 3
SB = 2
IB = 6
PER_TILE = E // NW
N_FULL = 104
N_LOOP = 102
CT = PER_TILE - N_FULL * C
N_PAD = 10112
RPT = N_PAD // NS

_PERM = np.empty((D,), dtype=np.int32)
for _k in range(D // 32):
    for _i in range(16):
        _PERM[32 * _k + 2 * _i] = 32 * _k + _i
        _PERM[32 * _k + 2 * _i + 1] = 32 * _k + 16 + _i


def _mm_body(h_ref, norm_ref, w_ref, o_ref):
    o_ref[...] = jnp.dot(h_ref[...] * norm_ref[...], w_ref[...],
                         preferred_element_type=jnp.float32
                         ).astype(jnp.bfloat16)


def _matmul(h, norm, w):
    grid = (5,)
    blk = N // grid[0]
    return pl.pallas_call(
        _mm_body,
        grid=grid,
        in_specs=[
            pl.BlockSpec((blk, D), lambda i: (i, 0)),
            pl.BlockSpec((blk, 1), lambda i: (i, 0)),
            pl.BlockSpec((D, D), lambda i: (0, 0)),
        ],
        out_specs=pl.BlockSpec((blk, D), lambda i: (i, 0)),
        out_shape=jax.ShapeDtypeStruct((N, D), jnp.bfloat16),
    )(h, norm, w)


def _sc_body(g_hbm, ei_hbm, ew_hbm, out_hbm,
             src_v, dst_v, ew_v, gb, sbuf, tsrc, tdst, tew,
             acc, isem, gsem, ssem, tsem):
    c = lax.axis_index("c")
    s = lax.axis_index("s")
    wid = c * NS + s
    tbase = wid * PER_TILE

    def start_idx(i, q):
        base = tbase + i * C
        pltpu.async_copy(ei_hbm.at[pl.ds(base, C)], src_v[q], isem[q])
        pltpu.async_copy(ei_hbm.at[pl.ds(E + base, C)], dst_v[q], isem[q])
        pltpu.async_copy(ew_hbm.at[pl.ds(base, C)], ew_v[q], isem[q])

    def wait_idx(i, q):
        base = tbase + i * C
        pltpu.make_async_copy(ei_hbm.at[pl.ds(base, C)], src_v[q], isem[q]).wait()
        pltpu.make_async_copy(ei_hbm.at[pl.ds(E + base, C)], dst_v[q], isem[q]).wait()
        pltpu.make_async_copy(ew_hbm.at[pl.ds(base, C)], ew_v[q], isem[q]).wait()

    def start_gather(r, q):
        pltpu.async_copy(g_hbm.at[src_v[q]], gb[r], gsem[r])

    def wait_gather(r, q):
        pltpu.make_async_copy(g_hbm.at[src_v[q]], gb[r], gsem[r]).wait()

    def start_scatter(b, q):
        pltpu.async_copy(sbuf[b], acc.at[dst_v[q]], ssem[b], add=True)

    def wait_scatter(b, q):
        pltpu.make_async_copy(sbuf[b], acc.at[dst_v[q]], ssem[b]).wait()

    start_idx(0, 0)
    start_idx(1, 1)

    zv = jnp.zeros((L,), jnp.float32)

    def zrow(r, _):
        for k in range(D // L):
            sbuf[0][r, pl.ds(k * L, L)] = zv
        return 0
    lax.fori_loop(0, C, zrow, 0)
    for off in range(0, RPT, C):
        sz = min(C, RPT - off)
        pltpu.sync_copy(sbuf[0].at[pl.ds(0, sz)],
                        acc.at[pl.ds(s * RPT + off, sz)])
    plsc.subcore_barrier()

    wait_idx(0, 0)
    start_gather(0, 0)

    def scale(gref, sref, wref, ngroups):
        def sgroup(g, _):
            wv = wref[pl.ds(g * L, L)]
            for jj in range(L):
                w = wv[jj]
                r = g * L + jj
                for k in range(D // 32):
                    word = gref[r, pl.ds(k * L, L)]
                    a = lax.bitcast_convert_type(word << 16, jnp.float32)
                    b = lax.bitcast_convert_type(
                        word & jnp.int32(-65536), jnp.float32)
                    sref[r, pl.ds(k * 32, L)] = a * w
                    sref[r, pl.ds(k * 32 + L, L)] = b * w
            return 0
        lax.fori_loop(0, ngroups, sgroup, 0)

    def chunk_body(i, j, wait_scat, do_idx, do_gather):
        if wait_scat:
            wait_scatter(j % SB, (j + 4) % IB)
        if do_idx:
            start_idx(i + 2, (j + 2) % IB)
        if do_gather:
            wait_idx(i + 1, (j + 1) % IB)
            start_gather((j + 1) % GB, (j + 1) % IB)
        wait_gather(j % GB, j % IB)
        scale(gb[j % GB], sbuf[j % SB], ew_v[j % IB], C // L)
        start_scatter(j % SB, j % IB)

    def group(gi, carry):
        for j in range(IB):
            i = gi * IB + j
            if j < 2:
                @pl.when(i >= 2)
                def _():
                    wait_scatter(j % SB, (j + 4) % IB)
                chunk_body(i, j, False, True, True)
            else:
                chunk_body(i, j, True, True, True)
        return carry

    lax.fori_loop(0, N_LOOP // IB, group, 0)

    for i in range(N_LOOP, N_FULL):
        j = i % IB
        chunk_body(i, j, True, i + 2 < N_FULL, i + 1 < N_FULL)

    wait_scatter((N_FULL - 2) % SB, (N_FULL - 2) % IB)
    wait_scatter((N_FULL - 1) % SB, (N_FULL - 1) % IB)

    tb = tbase + N_FULL * C
    pltpu.async_copy(ei_hbm.at[pl.ds(tb, CT)], tsrc, tsem)
    pltpu.async_copy(ei_hbm.at[pl.ds(E + tb, CT)], tdst, tsem)
    pltpu.async_copy(ew_hbm.at[pl.ds(tb, CT)], tew, tsem)
    pltpu.make_async_copy(ei_hbm.at[pl.ds(tb, CT)], tsrc, tsem).wait()
    pltpu.make_async_copy(ei_hbm.at[pl.ds(E + tb, CT)], tdst, tsem).wait()
    pltpu.make_async_copy(ew_hbm.at[pl.ds(tb, CT)], tew, tsem).wait()
    pltpu.async_copy(g_hbm.at[tsrc], gb[0].at[pl.ds(0, CT)], tsem)
    pltpu.make_async_copy(g_hbm.at[tsrc], gb[0].at[pl.ds(0, CT)], tsem).wait()
    scale(gb[0], sbuf[0], tew, CT // L)
    pltpu.async_copy(sbuf[0].at[pl.ds(0, CT)], acc.at[tdst], tsem, add=True)
    pltpu.make_async_copy(sbuf[0].at[pl.ds(0, CT)], acc.at[tdst], tsem).wait()
    plsc.subcore_barrier()

    pltpu.sync_copy(acc.at[pl.ds(s * RPT, RPT)],
                    out_hbm.at[c, pl.ds(s * RPT, RPT)])


_sc_edge_kernel = functools.partial(
    pl.kernel,
    out_type=jax.ShapeDtypeStruct((NC, N_PAD, D), jnp.float32),
    mesh=plsc.VectorSubcoreMesh(core_axis_name="c", subcore_axis_name="s"),
    compiler_params=pltpu.CompilerParams(use_tc_tiling_on_sc=False),
    scratch_types=[
        [pltpu.VMEM((C,), jnp.int32) for _ in range(IB)],
        [pltpu.VMEM((C,), jnp.int32) for _ in range(IB)],
        [pltpu.VMEM((C,), jnp.float32) for _ in range(IB)],
        [pltpu.VMEM((C, D // 2), jnp.int32) for _ in range(GB)],
        [pltpu.VMEM((C, D), jnp.float32) for _ in range(SB)],
        pltpu.VMEM((CT,), jnp.int32),
        pltpu.VMEM((CT,), jnp.int32),
        pltpu.VMEM((CT,), jnp.float32),
        pltpu.VMEM_SHARED((N_PAD, D), jnp.float32),
        [pltpu.SemaphoreType.DMA for _ in range(IB)],
        [pltpu.SemaphoreType.DMA for _ in range(GB)],
        [pltpu.SemaphoreType.DMA for _ in range(SB)],
        pltpu.SemaphoreType.DMA,
    ],
)(_sc_body)


def _comb_body(p0_ref, p1_ref, norm_ref, b_ref, o_ref):
    o_ref[...] = (p0_ref[0] + p1_ref[0]) * norm_ref[...] + b_ref[...]


def _combine(partials, norm, bias):
    grid = (5,)
    blk = N // grid[0]
    return pl.pallas_call(
        _comb_body,
        grid=grid,
        in_specs=[
            pl.BlockSpec((1, blk, D), lambda i: (0, i, 0)),
            pl.BlockSpec((1, blk, D), lambda i: (1, i, 0)),
            pl.BlockSpec((blk, 1), lambda i: (i, 0)),
            pl.BlockSpec((1, D), lambda i: (0, 0)),
        ],
        out_specs=pl.BlockSpec((blk, D), lambda i: (i, 0)),
        out_shape=jax.ShapeDtypeStruct((N, D), jnp.float32),
    )(partials, partials, norm, bias)


def kernel(h, edge_index, edge_weight, norm, weight, bias):
    g = _matmul(h, norm, jnp.take(weight, jnp.asarray(_PERM), axis=1))
    g32 = lax.bitcast_convert_type(g.reshape(N, D // 2, 2), jnp.int32)
    partials = _sc_edge_kernel(g32, edge_index.reshape(2 * E), edge_weight)
    return _combine(partials, norm, bias.reshape(1, D))


# speedup vs baseline: 2.0907x; 2.0907x over previous
import functools

import jax
import jax.numpy as jnp
from jax import lax
from jax.experimental import pallas as pl
from jax.experimental.pallas import tpu as pltpu
from jax.experimental.pallas import tpu_sc as plsc

N = 10000
E = 320000
D = 128

NC = 2
NS = 16
L = 16
NW = NC * NS

C = 112
RB = 3
IB = 6
PER_TILE = E // NW
N_FULL = 89
N_LOOP = 84
CT = PER_TILE - N_FULL * C
N_PAD = 10112
RPT = N_PAD // NS


def _mm_body(h_ref, norm_ref, w_ref, o_ref):
    o_ref[...] = jnp.dot(h_ref[...] * norm_ref[...], w_ref[...],
                         preferred_element_type=jnp.float32)


def _matmul(h, norm, w):
    grid = (2,)
    blk = N // grid[0]
    return pl.pallas_call(
        _mm_body,
        grid=grid,
        in_specs=[
            pl.BlockSpec((blk, D), lambda i: (i, 0)),
            pl.BlockSpec((blk, 1), lambda i: (i, 0)),
            pl.BlockSpec((D, D), lambda i: (0, 0)),
        ],
        out_specs=pl.BlockSpec((blk, D), lambda i: (i, 0)),
        out_shape=jax.ShapeDtypeStruct((N, D), jnp.float32),
    )(h, norm, w)


def _sc_body(g_hbm, ei_hbm, ew_hbm, out_hbm,
             src_v, dst_v, ew_v, rows, tsrc, tdst, tew, trows,
             acc, isem, gsem, ssem, tsem):
    c = lax.axis_index("c")
    s = lax.axis_index("s")
    wid = c * NS + s
    tbase = wid * PER_TILE

    def start_idx(i, q):
        base = tbase + i * C
        pltpu.async_copy(ei_hbm.at[pl.ds(base, C)], src_v[q], isem[q])
        pltpu.async_copy(ei_hbm.at[pl.ds(E + base, C)], dst_v[q], isem[q])
        pltpu.async_copy(ew_hbm.at[pl.ds(base, C)], ew_v[q], isem[q])

    def wait_idx(i, q):
        base = tbase + i * C
        pltpu.make_async_copy(ei_hbm.at[pl.ds(base, C)], src_v[q], isem[q]).wait()
        pltpu.make_async_copy(ei_hbm.at[pl.ds(E + base, C)], dst_v[q], isem[q]).wait()
        pltpu.make_async_copy(ew_hbm.at[pl.ds(base, C)], ew_v[q], isem[q]).wait()

    def start_gather(r, q):
        pltpu.async_copy(g_hbm.at[src_v[q]], rows[r], gsem[r])

    def wait_gather(r, q):
        pltpu.make_async_copy(g_hbm.at[src_v[q]], rows[r], gsem[r]).wait()

    def start_scatter(r, q):
        pltpu.async_copy(rows[r], acc.at[dst_v[q]], ssem[r], add=True)

    def wait_scatter(r, q):
        pltpu.make_async_copy(rows[r], acc.at[dst_v[q]], ssem[r]).wait()

    start_idx(0, 0)
    start_idx(1, 1)

    zv = jnp.zeros((L,), jnp.float32)

    def zrow(r, _):
        for k in range(D // L):
            rows[2][r, pl.ds(k * L, L)] = zv
        return 0
    lax.fori_loop(0, C, zrow, 0)
    wait_idx(0, 0)
    start_gather(0, 0)
    for off in range(0, RPT, C):
        sz = min(C, RPT - off)
        pltpu.sync_copy(rows[2].at[pl.ds(0, sz)],
                        acc.at[pl.ds(s * RPT + off, sz)])
    plsc.subcore_barrier()

    def scale(buf, wref, ngroups):
        def sgroup(g, _):
            wv = wref[pl.ds(g * L, L)]
            for jj in range(L):
                w = wv[jj]
                r = g * L + jj
                for k in range(D // L):
                    buf[r, pl.ds(k * L, L)] = buf[r, pl.ds(k * L, L)] * w
            return 0
        lax.fori_loop(0, ngroups, sgroup, 0)

    def chunk_body(i, j, wait_scat, do_idx, do_gather):
        if wait_scat:
            wait_scatter((j + 1) % RB, (j + 4) % IB)
        if do_idx:
            start_idx(i + 2, (j + 2) % IB)
        if do_gather:
            wait_idx(i + 1, (j + 1) % IB)
            start_gather((j + 1) % RB, (j + 1) % IB)
        wait_gather(j % RB, j % IB)
        scale(rows[j % RB], ew_v[j % IB], C // L)
        start_scatter(j % RB, j % IB)

    def group(gi, carry):
        for j in range(IB):
            i = gi * IB + j
            if j < 2:
                @pl.when(i >= 2)
                def _():
                    wait_scatter((j + 1) % RB, (j + 4) % IB)
                chunk_body(i, j, False, True, True)
            else:
                chunk_body(i, j, True, True, True)
        return carry

    lax.fori_loop(0, N_LOOP // IB, group, 0)

    for i in range(N_LOOP, N_FULL):
        j = i % IB
        chunk_body(i, j, True, i + 2 < N_FULL, i + 1 < N_FULL)

    tb = tbase + N_FULL * C
    pltpu.async_copy(ei_hbm.at[pl.ds(tb, CT)], tsrc, tsem)
    pltpu.async_copy(ei_hbm.at[pl.ds(E + tb, CT)], tdst, tsem)
    pltpu.async_copy(ew_hbm.at[pl.ds(tb, CT)], tew, tsem)
    pltpu.make_async_copy(ei_hbm.at[pl.ds(tb, CT)], tsrc, tsem).wait()
    pltpu.make_async_copy(ei_hbm.at[pl.ds(E + tb, CT)], tdst, tsem).wait()
    pltpu.make_async_copy(ew_hbm.at[pl.ds(tb, CT)], tew, tsem).wait()
    pltpu.async_copy(g_hbm.at[tsrc], trows, tsem)
    pltpu.make_async_copy(g_hbm.at[tsrc], trows, tsem).wait()
    scale(trows, tew, CT // L)
    pltpu.async_copy(trows, acc.at[tdst], tsem, add=True)
    pltpu.make_async_copy(trows, acc.at[tdst], tsem).wait()

    wait_scatter((N_FULL - 2) % RB, (N_FULL - 2) % IB)
    wait_scatter((N_FULL - 1) % RB, (N_FULL - 1) % IB)
    plsc.subcore_barrier()

    pltpu.sync_copy(acc.at[pl.ds(s * RPT, RPT)],
                    out_hbm.at[c, pl.ds(s * RPT, RPT)])


_sc_edge_kernel = functools.partial(
    pl.kernel,
    out_type=jax.ShapeDtypeStruct((NC, N_PAD, D), jnp.float32),
    mesh=plsc.VectorSubcoreMesh(core_axis_name="c", subcore_axis_name="s"),
    scratch_types=[
        [pltpu.VMEM((C,), jnp.int32) for _ in range(IB)],
        [pltpu.VMEM((C,), jnp.int32) for _ in range(IB)],
        [pltpu.VMEM((C,), jnp.float32) for _ in range(IB)],
        [pltpu.VMEM((C, D), jnp.float32) for _ in range(RB)],
        pltpu.VMEM((CT,), jnp.int32),
        pltpu.VMEM((CT,), jnp.int32),
        pltpu.VMEM((CT,), jnp.float32),
        pltpu.VMEM((CT, D), jnp.float32),
        pltpu.VMEM_SHARED((N_PAD, D), jnp.float32),
        [pltpu.SemaphoreType.DMA for _ in range(IB)],
        [pltpu.SemaphoreType.DMA for _ in range(RB)],
        [pltpu.SemaphoreType.DMA for _ in range(RB)],
        pltpu.SemaphoreType.DMA,
    ],
)(_sc_body)


def _comb_body(p0_ref, p1_ref, norm_ref, b_ref, o_ref):
    o_ref[...] = (p0_ref[0] + p1_ref[0]) * norm_ref[...] + b_ref[...]


def _combine(partials, norm, bias):
    grid = (2,)
    blk = N // grid[0]
    return pl.pallas_call(
        _comb_body,
        grid=grid,
        in_specs=[
            pl.BlockSpec((1, blk, D), lambda i: (0, i, 0)),
            pl.BlockSpec((1, blk, D), lambda i: (1, i, 0)),
            pl.BlockSpec((blk, 1), lambda i: (i, 0)),
            pl.BlockSpec((1, D), lambda i: (0, 0)),
        ],
        out_specs=pl.BlockSpec((blk, D), lambda i: (i, 0)),
        out_shape=jax.ShapeDtypeStruct((N, D), jnp.float32),
    )(partials, partials, norm, bias)


def kernel(h, edge_index, edge_weight, norm, weight, bias):
    g = _matmul(h, norm, weight)
    partials = _sc_edge_kernel(g, edge_index.reshape(2 * E), edge_weight)
    return _combine(partials, norm, bias.reshape(1, D))
